# Initial kernel scaffold; baseline (speedup 1.0000x reference)
import jax
import jax.numpy as jnp
from jax.experimental import pallas as pl


def kernel(pt_2d, color, pt_3d, normal, R, T, face):
    raise NotImplementedError("write your pallas kernel here")


# speedup vs baseline: 11.5567x; 11.5567x over previous
import functools

import jax
import jax.numpy as jnp
import numpy as np
from jax import lax
from jax.experimental import pallas as pl
from jax.experimental.pallas import tpu as pltpu
from jax.experimental.pallas import tpu_sc as plsc

FTINY = float(np.finfo(np.float32).tiny) * 1e3
INF_VALUE = float(np.finfo(np.float32).max) * 1e-3
LOWER_INF = float(np.finfo(np.float32).max) * 1e-4
H = 384
W = 384
BLK = 32
B = 2
V = 66
VP = 80
F = 64
L = 16
NC = 2
NS = 16
NW = NC * NS
ROWS_PW = (B * H) // NW
NCHUNK = W // L
PXW = ROWS_PW * W

R_L0X, R_L0Y, R_L0C = 0, 1, 2
R_L1X, R_L1Y, R_L1C = 3, 4, 5
R_L2X, R_L2Y, R_L2C = 6, 7, 8
R_DX, R_DY, R_DC = 9, 10, 11
R_CX = 12
R_CY = 15
R_CC = 18
R_PXMIN, R_PXMAX, R_PYMIN, R_PYMAX = 21, 22, 23, 24
R_VALID = 25
NCOEF = 26


def _splat(val, dtype=jnp.float32):
    return jnp.full((L,), val, dtype=dtype)


def _rasterize_body(pt2_h, col_h, pt3_h, nrm_h, rt_h, face_h,
                    img_h, msk_h,
                    pt2_v, col_v, pt3_v, nrm_v, rt_v, face_v,
                    coef_v, best_d, best_i, img_v, msk_v):
    wid = lax.axis_index("s") * NC + lax.axis_index("c")
    grow0 = wid * ROWS_PW
    b = grow0 // H
    row0 = grow0 % H

    pltpu.sync_copy(pt2_h, pt2_v)
    pltpu.sync_copy(col_h, col_v)
    pltpu.sync_copy(pt3_h, pt3_v)
    pltpu.sync_copy(nrm_h, nrm_v)
    pltpu.sync_copy(rt_h, rt_v)
    pltpu.sync_copy(face_h, face_v)

    iota_i = lax.iota(jnp.int32, L)
    vb3 = _splat(b * 3 * VP, jnp.int32)

    def rt_bcast(i):
        return plsc.load_gather(rt_v, [_splat(b * L + i, jnp.int32)])

    t_c = [
        rt_bcast(0 + c) * rt_bcast(9)
        + rt_bcast(3 + c) * rt_bcast(10)
        + rt_bcast(6 + c) * rt_bcast(11)
        for c in range(3)
    ]

    big_i = jnp.int32(2 ** 30)
    xmin_acc = _splat(big_i, jnp.int32)
    xmax_acc = _splat(-big_i, jnp.int32)
    ymin_acc = _splat(big_i, jnp.int32)
    ymax_acc = _splat(-big_i, jnp.int32)
    anyv_acc = jnp.zeros((L,), jnp.int32)
    for j in range(F // L):
        fi = [face_v[pl.ds(k * F + j * L, L)] for k in range(3)]

        def vgather(ref, c, idx):
            return plsc.load_gather(ref, [vb3 + c * VP + idx])

        p = [[vgather(pt2_v, c, fi[k]) for k in range(3)] for c in range(3)]
        nrm = [plsc.load_gather(
                   nrm_v, [_splat((b * 3 + c) * F + j * L, jnp.int32) + iota_i])
               for c in range(3)]
        s = sum((vgather(pt3_v, c, fi[0]) + t_c[c]) * nrm[c] for c in range(3))
        norm_cul = s < 0.0
        depth_cul = jnp.minimum(jnp.minimum(p[2][0], p[2][1]), p[2][2]) > 0.0
        valid = norm_cul & depth_cul

        det = ((p[1][1] - p[1][2]) * (p[0][0] - p[0][2])
               + (p[0][2] - p[0][1]) * (p[1][0] - p[1][2]))
        det = jnp.sign(det) * jnp.maximum(jnp.abs(det), FTINY)
        inv = 1.0 / det
        l0x = (p[1][1] - p[1][2]) * inv
        l0y = (p[0][2] - p[0][1]) * inv
        l0c = -l0x * p[0][2] - l0y * p[1][2]
        l1x = (p[1][2] - p[1][0]) * inv
        l1y = (p[0][0] - p[0][2]) * inv
        l1c = -l1x * p[0][2] - l1y * p[1][2]
        l2x = -l0x - l1x
        l2y = -l0y - l1y
        l2c = 1.0 - l0c - l1c

        def cput(row, vec):
            coef_v[pl.ds(row * F + j * L, L)] = vec

        cput(R_L0X, l0x)
        cput(R_L0Y, l0y)
        cput(R_L0C, l0c)
        cput(R_L1X, l1x)
        cput(R_L1Y, l1y)
        cput(R_L1C, l1c)
        cput(R_L2X, l2x)
        cput(R_L2Y, l2y)
        cput(R_L2C, l2c)
        cput(R_DX, p[2][0] * l0x + p[2][1] * l1x + p[2][2] * l2x)
        cput(R_DY, p[2][0] * l0y + p[2][1] * l1y + p[2][2] * l2y)
        cput(R_DC, p[2][0] * l0c + p[2][1] * l1c + p[2][2] * l2c)
        for ch in range(3):
            c0 = vgather(col_v, ch, fi[0])
            c1 = vgather(col_v, ch, fi[1])
            c2 = vgather(col_v, ch, fi[2])
            cput(R_CX + ch, c0 * l0x + c1 * l1x + c2 * l2x)
            cput(R_CY + ch, c0 * l0y + c1 * l1y + c2 * l2y)
            cput(R_CC + ch, c0 * l0c + c1 * l1c + c2 * l2c)

        pxi = [p[0][k].astype(jnp.int32) for k in range(3)]
        pyi = [p[1][k].astype(jnp.int32) for k in range(3)]
        px_min = jnp.minimum(jnp.minimum(pxi[0], pxi[1]), pxi[2])
        px_max = jnp.maximum(jnp.maximum(pxi[0], pxi[1]), pxi[2])
        py_min = jnp.minimum(jnp.minimum(pyi[0], pyi[1]), pyi[2])
        py_max = jnp.maximum(jnp.maximum(pyi[0], pyi[1]), pyi[2])
        cput(R_PXMIN, px_min.astype(jnp.float32))
        cput(R_PXMAX, px_max.astype(jnp.float32))
        cput(R_PYMIN, py_min.astype(jnp.float32))
        cput(R_PYMAX, py_max.astype(jnp.float32))
        cput(R_VALID, jnp.where(valid, 1.0, 0.0))

        xmin_acc = jnp.minimum(xmin_acc, jnp.where(valid, px_min, big_i))
        xmax_acc = jnp.maximum(xmax_acc, jnp.where(valid, px_max, -big_i))
        ymin_acc = jnp.minimum(ymin_acc, jnp.where(valid, py_min, big_i))
        ymax_acc = jnp.maximum(ymax_acc, jnp.where(valid, py_max, -big_i))
        anyv_acc = anyv_acc | valid.astype(jnp.int32)

    x_min = jnp.min(xmin_acc)
    x_max = jnp.max(xmax_acc)
    y_min = jnp.min(ymin_acc)
    y_max = jnp.max(ymax_acc)
    any_valid = jnp.max(anyv_acc) > 0
    range_x_min = jnp.maximum(x_min - lax.rem(x_min, BLK), 0)
    range_y_min = jnp.maximum(y_min - lax.rem(y_min, BLK), 0)
    range_x_max = jnp.minimum(x_max, W)
    range_y_max = jnp.minimum(y_max, H)

    iota_f = iota_i.astype(jnp.float32)

    def init_chunk(k, _):
        sl = pl.ds(k * L, L)
        best_d[sl] = _splat(jnp.float32(jnp.inf))
        best_i[sl] = _splat(0, jnp.int32)
        return 0

    lax.fori_loop(0, PXW // L, init_chunk, 0)

    def face_body(f, _):
        fi32 = _splat(f, jnp.int32)

        def cb(row):
            return plsc.load_gather(coef_v, [_splat(row * F, jnp.int32) + fi32])

        bl0x, bl0y, bl0c = cb(R_L0X), cb(R_L0Y), cb(R_L0C)
        bl1x, bl1y, bl1c = cb(R_L1X), cb(R_L1Y), cb(R_L1C)
        bl2x, bl2y, bl2c = cb(R_L2X), cb(R_L2Y), cb(R_L2C)
        bdx, bdy, bdc = cb(R_DX), cb(R_DY), cb(R_DC)
        bpxmin, bpxmax = cb(R_PXMIN), cb(R_PXMAX)
        bpymin, bpymax = cb(R_PYMIN), cb(R_PYMAX)
        bvalid = cb(R_VALID) > 0.5

        def row_body(r, _):
            y = row0 + r
            yf = _splat(y.astype(jnp.float32))
            ibyf = _splat((y - lax.rem(y, BLK)).astype(jnp.float32))
            l0yc = bl0y * yf + bl0c
            l1yc = bl1y * yf + bl1c
            l2yc = bl2y * yf + bl2c
            dyc = bdy * yf + bdc
            tgy = (bpymax >= ibyf) & (bpymin < ibyf + float(BLK))
            mrow = bvalid & tgy
            rbase = r * W

            def chunk_body(j, _):
                xi = iota_i + j * L
                xf = xi.astype(jnp.float32)
                kbxf = (xi - lax.rem(xi, BLK)).astype(jnp.float32)
                l0 = bl0x * xf + l0yc
                l1 = bl1x * xf + l1yc
                l2 = bl2x * xf + l2yc
                raw = bdx * xf + dyc
                tgx = (bpxmax >= kbxf) & (bpxmin < kbxf + float(BLK))
                inside = ((l0 >= 0.0) & (l1 >= 0.0) & (l2 >= 0.0)
                          & tgx & mrow)
                d = jnp.where(inside, raw, INF_VALUE + raw)
                d = jnp.where(d != d, INF_VALUE, d)
                sl = pl.ds(rbase + j * L, L)
                bd = best_d[sl]
                upd = d < bd
                best_d[sl] = jnp.where(upd, d, bd)
                best_i[sl] = jnp.where(upd, fi32, best_i[sl])
                return 0

            return lax.fori_loop(0, NCHUNK, chunk_body, 0)

        return lax.fori_loop(0, ROWS_PW, row_body, 0)

    lax.fori_loop(0, F, face_body, 0)

    rxminf = _splat(range_x_min.astype(jnp.float32))
    rxmaxf = _splat(range_x_max.astype(jnp.float32))
    ryminf = _splat(range_y_min.astype(jnp.float32))
    rymaxf = _splat(range_y_max.astype(jnp.float32))
    anyv_v = _splat(jnp.where(any_valid, 1.0, 0.0)) > 0.5

    def color_row(r, _):
        y = row0 + r
        yf = _splat(y.astype(jnp.float32))
        ibyf = _splat((y - lax.rem(y, BLK)).astype(jnp.float32))
        procy = (ibyf >= ryminf) & (ibyf < rymaxf)
        mrow = procy & anyv_v
        rbase = r * W

        def color_chunk(j, _):
            xi = iota_i + j * L
            xf = xi.astype(jnp.float32)
            kbxf = (xi - lax.rem(xi, BLK)).astype(jnp.float32)
            sl = pl.ds(rbase + j * L, L)
            bd = best_d[sl]
            bi = best_i[sl]
            vis = bd < LOWER_INF
            procx = (kbxf >= rxminf) & (kbxf < rxmaxf)
            covered = vis & procx & mrow
            for ch in range(3):
                cx = plsc.load_gather(
                    coef_v, [_splat((R_CX + ch) * F, jnp.int32) + bi])
                cy = plsc.load_gather(
                    coef_v, [_splat((R_CY + ch) * F, jnp.int32) + bi])
                cc = plsc.load_gather(
                    coef_v, [_splat((R_CC + ch) * F, jnp.int32) + bi])
                pix = cx * xf + cy * yf + cc
                img_v[pl.ds(ch * PXW + rbase + j * L, L)] = (
                    jnp.where(covered, pix, 0.0))
            msk_v[sl] = jnp.where(covered, 1.0, 0.0)
            return 0

        return lax.fori_loop(0, NCHUNK, color_chunk, 0)

    lax.fori_loop(0, ROWS_PW, color_row, 0)

    for ch in range(3):
        pltpu.sync_copy(
            img_v.at[pl.ds(ch * PXW, PXW)],
            img_h.at[pl.ds(((b * 3 + ch) * H + row0) * W, PXW)])
    pltpu.sync_copy(msk_v, msk_h.at[pl.ds((b * H + row0) * W, PXW)])


@jax.jit
def _rasterize(pt2p, colp, pt3p, nrmp, rtp, facep):
    mesh = plsc.VectorSubcoreMesh(core_axis_name="c", subcore_axis_name="s")
    run = functools.partial(
        pl.kernel,
        out_type=[
            jax.ShapeDtypeStruct((B * 3 * H * W,), jnp.float32),
            jax.ShapeDtypeStruct((B * H * W,), jnp.float32),
        ],
        mesh=mesh,
        compiler_params=pltpu.CompilerParams(needs_layout_passes=False),
        scratch_types=[
            pltpu.VMEM((B * 3 * VP,), jnp.float32),
            pltpu.VMEM((B * 3 * VP,), jnp.float32),
            pltpu.VMEM((B * 3 * VP,), jnp.float32),
            pltpu.VMEM((B * 3 * F,), jnp.float32),
            pltpu.VMEM((B * L,), jnp.float32),
            pltpu.VMEM((3 * F,), jnp.int32),
            pltpu.VMEM((NCOEF * F,), jnp.float32),
            pltpu.VMEM((PXW,), jnp.float32),
            pltpu.VMEM((PXW,), jnp.int32),
            pltpu.VMEM((3 * PXW,), jnp.float32),
            pltpu.VMEM((PXW,), jnp.float32),
        ],
    )(_rasterize_body)
    img_flat, msk_flat = run(pt2p, colp, pt3p, nrmp, rtp, facep)
    image = img_flat.reshape(B, 3, H, W)
    mask = msk_flat.reshape(B, H, W)
    return image, mask


def kernel(pt_2d, color, pt_3d, normal, R, T, face):
    pad = ((0, 0), (0, 0), (0, VP - V))
    pt2p = jnp.pad(pt_2d, pad).reshape(-1)
    colp = jnp.pad(color, pad).reshape(-1)
    pt3p = jnp.pad(pt_3d, pad).reshape(-1)
    rtp = jnp.concatenate(
        [R.reshape(B, 9), T.reshape(B, 3), jnp.zeros((B, 4), jnp.float32)],
        axis=1).reshape(-1)
    return _rasterize(pt2p, colp, pt3p, normal.reshape(-1), rtp,
                      face.astype(jnp.int32).reshape(-1))


# speedup vs baseline: 12.5569x; 1.0866x over previous
import functools

import jax
import jax.numpy as jnp
import numpy as np
from jax import lax
from jax.experimental import pallas as pl
from jax.experimental.pallas import tpu as pltpu
from jax.experimental.pallas import tpu_sc as plsc

FTINY = float(np.finfo(np.float32).tiny) * 1e3
INF_VALUE = float(np.finfo(np.float32).max) * 1e-3
LOWER_INF = float(np.finfo(np.float32).max) * 1e-4
H = 384
W = 384
BLK = 32
B = 2
V = 66
VP = 80
F = 64
L = 16
NC = 2
NS = 16
NW = NC * NS
ROWS_PW = (B * H) // NW
NCHUNK = W // L
PXW = ROWS_PW * W

R_L0X, R_L0Y, R_L0C = 0, 1, 2
R_L1X, R_L1Y, R_L1C = 3, 4, 5
R_L2X, R_L2Y, R_L2C = 6, 7, 8
R_DX, R_DY, R_DC = 9, 10, 11
R_CX = 12
R_CY = 15
R_CC = 18
R_PXMIN, R_PXMAX, R_PYMIN, R_PYMAX = 21, 22, 23, 24
R_VALID = 25
NCOEF = 26


def _splat(val, dtype=jnp.float32):
    return jnp.full((L,), val, dtype=dtype)


def _rasterize_body(pt2_h, col_h, pt3_h, nrm_h, rt_h, face_h,
                    img_h, msk_h,
                    pt2_v, col_v, pt3_v, nrm_v, rt_v, face_v,
                    coef_v, best_d, best_i, img_v, msk_v):
    wid = lax.axis_index("s") * NC + lax.axis_index("c")
    grow0 = wid * ROWS_PW
    b = grow0 // H
    row0 = grow0 % H

    pltpu.sync_copy(pt2_h, pt2_v)
    pltpu.sync_copy(col_h, col_v)
    pltpu.sync_copy(pt3_h, pt3_v)
    pltpu.sync_copy(nrm_h, nrm_v)
    pltpu.sync_copy(rt_h, rt_v)
    pltpu.sync_copy(face_h, face_v)

    iota_i = lax.iota(jnp.int32, L)
    vb3 = _splat(b * 3 * VP, jnp.int32)

    def rt_bcast(i):
        return plsc.load_gather(rt_v, [_splat(b * L + i, jnp.int32)])

    t_c = [
        rt_bcast(0 + c) * rt_bcast(9)
        + rt_bcast(3 + c) * rt_bcast(10)
        + rt_bcast(6 + c) * rt_bcast(11)
        for c in range(3)
    ]

    big_i = jnp.int32(2 ** 30)
    xmin_acc = _splat(big_i, jnp.int32)
    xmax_acc = _splat(-big_i, jnp.int32)
    ymin_acc = _splat(big_i, jnp.int32)
    ymax_acc = _splat(-big_i, jnp.int32)
    anyv_acc = jnp.zeros((L,), jnp.int32)
    for j in range(F // L):
        fi = [face_v[pl.ds(k * F + j * L, L)] for k in range(3)]

        def vgather(ref, c, idx):
            return plsc.load_gather(ref, [vb3 + c * VP + idx])

        p = [[vgather(pt2_v, c, fi[k]) for k in range(3)] for c in range(3)]
        nrm = [plsc.load_gather(
                   nrm_v, [_splat((b * 3 + c) * F + j * L, jnp.int32) + iota_i])
               for c in range(3)]
        s = sum((vgather(pt3_v, c, fi[0]) + t_c[c]) * nrm[c] for c in range(3))
        norm_cul = s < 0.0
        depth_cul = jnp.minimum(jnp.minimum(p[2][0], p[2][1]), p[2][2]) > 0.0
        valid = norm_cul & depth_cul

        det = ((p[1][1] - p[1][2]) * (p[0][0] - p[0][2])
               + (p[0][2] - p[0][1]) * (p[1][0] - p[1][2]))
        det = jnp.sign(det) * jnp.maximum(jnp.abs(det), FTINY)
        inv = 1.0 / det
        l0x = (p[1][1] - p[1][2]) * inv
        l0y = (p[0][2] - p[0][1]) * inv
        l0c = -l0x * p[0][2] - l0y * p[1][2]
        l1x = (p[1][2] - p[1][0]) * inv
        l1y = (p[0][0] - p[0][2]) * inv
        l1c = -l1x * p[0][2] - l1y * p[1][2]
        l2x = -l0x - l1x
        l2y = -l0y - l1y
        l2c = 1.0 - l0c - l1c

        def cput(row, vec):
            coef_v[pl.ds(row * F + j * L, L)] = vec

        cput(R_L0X, l0x)
        cput(R_L0Y, l0y)
        cput(R_L0C, l0c)
        cput(R_L1X, l1x)
        cput(R_L1Y, l1y)
        cput(R_L1C, l1c)
        cput(R_L2X, l2x)
        cput(R_L2Y, l2y)
        cput(R_L2C, l2c)
        cput(R_DX, p[2][0] * l0x + p[2][1] * l1x + p[2][2] * l2x)
        cput(R_DY, p[2][0] * l0y + p[2][1] * l1y + p[2][2] * l2y)
        cput(R_DC, p[2][0] * l0c + p[2][1] * l1c + p[2][2] * l2c)
        for ch in range(3):
            c0 = vgather(col_v, ch, fi[0])
            c1 = vgather(col_v, ch, fi[1])
            c2 = vgather(col_v, ch, fi[2])
            cput(R_CX + ch, c0 * l0x + c1 * l1x + c2 * l2x)
            cput(R_CY + ch, c0 * l0y + c1 * l1y + c2 * l2y)
            cput(R_CC + ch, c0 * l0c + c1 * l1c + c2 * l2c)

        pxi = [p[0][k].astype(jnp.int32) for k in range(3)]
        pyi = [p[1][k].astype(jnp.int32) for k in range(3)]
        px_min = jnp.minimum(jnp.minimum(pxi[0], pxi[1]), pxi[2])
        px_max = jnp.maximum(jnp.maximum(pxi[0], pxi[1]), pxi[2])
        py_min = jnp.minimum(jnp.minimum(pyi[0], pyi[1]), pyi[2])
        py_max = jnp.maximum(jnp.maximum(pyi[0], pyi[1]), pyi[2])
        cput(R_PXMIN, px_min.astype(jnp.float32))
        cput(R_PXMAX, px_max.astype(jnp.float32))
        cput(R_PYMIN, py_min.astype(jnp.float32))
        cput(R_PYMAX, py_max.astype(jnp.float32))
        cput(R_VALID, jnp.where(valid, 1.0, 0.0))

        xmin_acc = jnp.minimum(xmin_acc, jnp.where(valid, px_min, big_i))
        xmax_acc = jnp.maximum(xmax_acc, jnp.where(valid, px_max, -big_i))
        ymin_acc = jnp.minimum(ymin_acc, jnp.where(valid, py_min, big_i))
        ymax_acc = jnp.maximum(ymax_acc, jnp.where(valid, py_max, -big_i))
        anyv_acc = anyv_acc | valid.astype(jnp.int32)

    x_min = jnp.min(xmin_acc)
    x_max = jnp.max(xmax_acc)
    y_min = jnp.min(ymin_acc)
    y_max = jnp.max(ymax_acc)
    any_valid = jnp.max(anyv_acc) > 0
    range_x_min = jnp.maximum(x_min - lax.rem(x_min, BLK), 0)
    range_y_min = jnp.maximum(y_min - lax.rem(y_min, BLK), 0)
    range_x_max = jnp.minimum(x_max, W)
    range_y_max = jnp.minimum(y_max, H)

    iota_f = iota_i.astype(jnp.float32)

    def sweep_face(f, first):
        fi32 = _splat(f, jnp.int32)

        def cb(row):
            return plsc.load_gather(coef_v, [_splat(row * F, jnp.int32) + fi32])

        bdx, bdy, bdc = cb(R_DX), cb(R_DY), cb(R_DC)
        sdx = bdx * float(L)
        valid_s = jnp.max(cb(R_VALID)) > 0.5

        def update(j, rbase, d):
            d = jnp.where(d != d, INF_VALUE, d)
            sl = pl.ds(rbase + j * L, L)
            if first:
                best_d[sl] = d
                best_i[sl] = _splat(0, jnp.int32)
            else:
                bd = best_d[sl]
                upd = d < bd
                best_d[sl] = jnp.where(upd, d, bd)
                best_i[sl] = jnp.where(upd, fi32, best_i[sl])

        bvalid = cb(R_VALID) > 0.5

        def full_path():
            bl0x, bl0y, bl0c = cb(R_L0X), cb(R_L0Y), cb(R_L0C)
            bl1x, bl1y, bl1c = cb(R_L1X), cb(R_L1Y), cb(R_L1C)
            bl2x, bl2y, bl2c = cb(R_L2X), cb(R_L2Y), cb(R_L2C)
            bpxmin, bpxmax = cb(R_PXMIN), cb(R_PXMAX)
            bpymin, bpymax = cb(R_PYMIN), cb(R_PYMAX)
            s0, s1, s2 = bl0x * float(L), bl1x * float(L), bl2x * float(L)

            def row_body(r, _):
                y = row0 + r
                yf = _splat(y.astype(jnp.float32))
                ibyf = _splat((y - lax.rem(y, BLK)).astype(jnp.float32))
                tgy = (bpymax >= ibyf) & (bpymin < ibyf + float(BLK))
                l0yc = bl0y * yf + bl0c
                l1yc = bl1y * yf + bl1c
                l2yc = bl2y * yf + bl2c
                dyc = bdy * yf + bdc
                rbase = r * W

                def chunk_body(j, _):
                    xf = iota_f + _splat((j * L).astype(jnp.float32))
                    l0v = bl0x * xf + l0yc
                    l1v = bl1x * xf + l1yc
                    l2v = bl2x * xf + l2yc
                    rv = bdx * xf + dyc
                    kbxf = _splat(
                        ((j * L) - lax.rem(j * L, BLK)).astype(jnp.float32))
                    tgx = (bpxmax >= kbxf) & (bpxmin < kbxf + float(BLK))
                    inside = ((l0v >= 0.0) & (l1v >= 0.0)
                              & ((l2v >= 0.0) & tgx) & (tgy & bvalid))
                    d = jnp.where(inside, rv, INF_VALUE + rv)
                    update(j, rbase, d)
                    return 0

                lax.fori_loop(0, NCHUNK, chunk_body, 0)
                return 0

            lax.fori_loop(0, ROWS_PW, row_body, 0)

        def fast_path():
            def row_body(r, _):
                y = row0 + r
                yf = _splat(y.astype(jnp.float32))
                rv = bdx * iota_f + (bdy * yf + bdc)
                rbase = r * W

                def chunk_body(j, rv):
                    update(j, rbase, INF_VALUE + rv)
                    return rv + sdx

                lax.fori_loop(0, NCHUNK, chunk_body, rv)
                return 0

            lax.fori_loop(0, ROWS_PW, row_body, 0)

        del valid_s, fast_path
        full_path()

    sweep_face(jnp.int32(0), True)

    def face_body(f, _):
        sweep_face(f, False)
        return 0

    lax.fori_loop(1, F, face_body, 0)

    rxminf = _splat(range_x_min.astype(jnp.float32))
    rxmaxf = _splat(range_x_max.astype(jnp.float32))
    ryminf = _splat(range_y_min.astype(jnp.float32))
    rymaxf = _splat(range_y_max.astype(jnp.float32))
    anyv_v = _splat(jnp.where(any_valid, 1.0, 0.0)) > 0.5

    def color_row(r, _):
        y = row0 + r
        yf = _splat(y.astype(jnp.float32))
        ibyf = _splat((y - lax.rem(y, BLK)).astype(jnp.float32))
        procy = (ibyf >= ryminf) & (ibyf < rymaxf)
        mrow = procy & anyv_v
        rbase = r * W

        def color_chunk(j, _):
            xi = iota_i + j * L
            xf = xi.astype(jnp.float32)
            kbxf = (xi - lax.rem(xi, BLK)).astype(jnp.float32)
            sl = pl.ds(rbase + j * L, L)
            bd = best_d[sl]
            bi = best_i[sl]
            vis = bd < LOWER_INF
            procx = (kbxf >= rxminf) & (kbxf < rxmaxf)
            covered = vis & procx & mrow
            for ch in range(3):
                cx = plsc.load_gather(
                    coef_v, [_splat((R_CX + ch) * F, jnp.int32) + bi])
                cy = plsc.load_gather(
                    coef_v, [_splat((R_CY + ch) * F, jnp.int32) + bi])
                cc = plsc.load_gather(
                    coef_v, [_splat((R_CC + ch) * F, jnp.int32) + bi])
                pix = cx * xf + cy * yf + cc
                img_v[pl.ds(ch * PXW + rbase + j * L, L)] = (
                    jnp.where(covered, pix, 0.0))
            msk_v[sl] = jnp.where(covered, 1.0, 0.0)
            return 0

        return lax.fori_loop(0, NCHUNK, color_chunk, 0)

    lax.fori_loop(0, ROWS_PW, color_row, 0)

    for ch in range(3):
        pltpu.sync_copy(
            img_v.at[pl.ds(ch * PXW, PXW)],
            img_h.at[pl.ds(((b * 3 + ch) * H + row0) * W, PXW)])
    pltpu.sync_copy(msk_v, msk_h.at[pl.ds((b * H + row0) * W, PXW)])


@jax.jit
def _rasterize(pt2p, colp, pt3p, nrmp, rtp, facep):
    mesh = plsc.VectorSubcoreMesh(core_axis_name="c", subcore_axis_name="s")
    run = functools.partial(
        pl.kernel,
        out_type=[
            jax.ShapeDtypeStruct((B * 3 * H * W,), jnp.float32),
            jax.ShapeDtypeStruct((B * H * W,), jnp.float32),
        ],
        mesh=mesh,
        compiler_params=pltpu.CompilerParams(needs_layout_passes=False),
        scratch_types=[
            pltpu.VMEM((B * 3 * VP,), jnp.float32),
            pltpu.VMEM((B * 3 * VP,), jnp.float32),
            pltpu.VMEM((B * 3 * VP,), jnp.float32),
            pltpu.VMEM((B * 3 * F,), jnp.float32),
            pltpu.VMEM((B * L,), jnp.float32),
            pltpu.VMEM((3 * F,), jnp.int32),
            pltpu.VMEM((NCOEF * F,), jnp.float32),
            pltpu.VMEM((PXW,), jnp.float32),
            pltpu.VMEM((PXW,), jnp.int32),
            pltpu.VMEM((3 * PXW,), jnp.float32),
            pltpu.VMEM((PXW,), jnp.float32),
        ],
    )(_rasterize_body)
    img_flat, msk_flat = run(pt2p, colp, pt3p, nrmp, rtp, facep)
    image = img_flat.reshape(B, 3, H, W)
    mask = msk_flat.reshape(B, H, W)
    return image, mask


def kernel(pt_2d, color, pt_3d, normal, R, T, face):
    pad = ((0, 0), (0, 0), (0, VP - V))
    pt2p = jnp.pad(pt_2d, pad).reshape(-1)
    colp = jnp.pad(color, pad).reshape(-1)
    pt3p = jnp.pad(pt_3d, pad).reshape(-1)
    rtp = jnp.concatenate(
        [R.reshape(B, 9), T.reshape(B, 3), jnp.zeros((B, 4), jnp.float32)],
        axis=1).reshape(-1)
    return _rasterize(pt2p, colp, pt3p, normal.reshape(-1), rtp,
                      face.astype(jnp.int32).reshape(-1))


# speedup vs baseline: 14.5255x; 1.1568x over previous
import functools

import jax
import jax.numpy as jnp
import numpy as np
from jax import lax
from jax.experimental import pallas as pl
from jax.experimental.pallas import tpu as pltpu
from jax.experimental.pallas import tpu_sc as plsc

FTINY = float(np.finfo(np.float32).tiny) * 1e3
INF_VALUE = float(np.finfo(np.float32).max) * 1e-3
LOWER_INF = float(np.finfo(np.float32).max) * 1e-4
H = 384
W = 384
BLK = 32
B = 2
V = 66
VP = 80
F = 64
L = 16
NC = 2
NS = 16
NW = NC * NS
ROWS_PW = (B * H) // NW
NCHUNK = W // L
PXW = ROWS_PW * W

R_L0X, R_L0Y, R_L0C = 0, 1, 2
R_L1X, R_L1Y, R_L1C = 3, 4, 5
R_L2X, R_L2Y, R_L2C = 6, 7, 8
R_DX, R_DY, R_DC = 9, 10, 11
R_CX = 12
R_CY = 15
R_CC = 18
R_PXMIN, R_PXMAX, R_PYMIN, R_PYMAX = 21, 22, 23, 24
R_VALID = 25
NCOEF = 26


def _splat(val, dtype=jnp.float32):
    return jnp.full((L,), val, dtype=dtype)


def _rasterize_body(pt2_h, col_h, pt3_h, nrm_h, rt_h, face_h,
                    img_h, msk_h,
                    pt2_v, col_v, pt3_v, nrm_v, rt_v, face_v,
                    coef_v, best_d, best_i, img_v, msk_v):
    wid = lax.axis_index("s") * NC + lax.axis_index("c")
    grow0 = wid * ROWS_PW
    b = grow0 // H
    row0 = grow0 % H

    pltpu.sync_copy(pt2_h, pt2_v)
    pltpu.sync_copy(col_h, col_v)
    pltpu.sync_copy(pt3_h, pt3_v)
    pltpu.sync_copy(nrm_h, nrm_v)
    pltpu.sync_copy(rt_h, rt_v)
    pltpu.sync_copy(face_h, face_v)

    iota_i = lax.iota(jnp.int32, L)
    vb3 = _splat(b * 3 * VP, jnp.int32)

    def rt_bcast(i):
        return plsc.load_gather(rt_v, [_splat(b * L + i, jnp.int32)])

    t_c = [
        rt_bcast(0 + c) * rt_bcast(9)
        + rt_bcast(3 + c) * rt_bcast(10)
        + rt_bcast(6 + c) * rt_bcast(11)
        for c in range(3)
    ]

    big_i = jnp.int32(2 ** 30)
    xmin_acc = _splat(big_i, jnp.int32)
    xmax_acc = _splat(-big_i, jnp.int32)
    ymin_acc = _splat(big_i, jnp.int32)
    ymax_acc = _splat(-big_i, jnp.int32)
    anyv_acc = jnp.zeros((L,), jnp.int32)
    for j in range(F // L):
        fi = [face_v[pl.ds(k * F + j * L, L)] for k in range(3)]

        def vgather(ref, c, idx):
            return plsc.load_gather(ref, [vb3 + c * VP + idx])

        p = [[vgather(pt2_v, c, fi[k]) for k in range(3)] for c in range(3)]
        nrm = [plsc.load_gather(
                   nrm_v, [_splat((b * 3 + c) * F + j * L, jnp.int32) + iota_i])
               for c in range(3)]
        s = sum((vgather(pt3_v, c, fi[0]) + t_c[c]) * nrm[c] for c in range(3))
        norm_cul = s < 0.0
        depth_cul = jnp.minimum(jnp.minimum(p[2][0], p[2][1]), p[2][2]) > 0.0
        valid = norm_cul & depth_cul

        det = ((p[1][1] - p[1][2]) * (p[0][0] - p[0][2])
               + (p[0][2] - p[0][1]) * (p[1][0] - p[1][2]))
        det = jnp.sign(det) * jnp.maximum(jnp.abs(det), FTINY)
        inv = 1.0 / det
        l0x = (p[1][1] - p[1][2]) * inv
        l0y = (p[0][2] - p[0][1]) * inv
        l0c = -l0x * p[0][2] - l0y * p[1][2]
        l1x = (p[1][2] - p[1][0]) * inv
        l1y = (p[0][0] - p[0][2]) * inv
        l1c = -l1x * p[0][2] - l1y * p[1][2]
        l2x = -l0x - l1x
        l2y = -l0y - l1y
        l2c = 1.0 - l0c - l1c

        def cput(row, vec):
            coef_v[pl.ds(row * F + j * L, L)] = vec

        cput(R_L0X, l0x)
        cput(R_L0Y, l0y)
        cput(R_L0C, l0c)
        cput(R_L1X, l1x)
        cput(R_L1Y, l1y)
        cput(R_L1C, l1c)
        cput(R_L2X, l2x)
        cput(R_L2Y, l2y)
        cput(R_L2C, l2c)
        cput(R_DX, p[2][0] * l0x + p[2][1] * l1x + p[2][2] * l2x)
        cput(R_DY, p[2][0] * l0y + p[2][1] * l1y + p[2][2] * l2y)
        cput(R_DC, p[2][0] * l0c + p[2][1] * l1c + p[2][2] * l2c)
        for ch in range(3):
            c0 = vgather(col_v, ch, fi[0])
            c1 = vgather(col_v, ch, fi[1])
            c2 = vgather(col_v, ch, fi[2])
            cput(R_CX + ch, c0 * l0x + c1 * l1x + c2 * l2x)
            cput(R_CY + ch, c0 * l0y + c1 * l1y + c2 * l2y)
            cput(R_CC + ch, c0 * l0c + c1 * l1c + c2 * l2c)

        pxi = [p[0][k].astype(jnp.int32) for k in range(3)]
        pyi = [p[1][k].astype(jnp.int32) for k in range(3)]
        px_min = jnp.minimum(jnp.minimum(pxi[0], pxi[1]), pxi[2])
        px_max = jnp.maximum(jnp.maximum(pxi[0], pxi[1]), pxi[2])
        py_min = jnp.minimum(jnp.minimum(pyi[0], pyi[1]), pyi[2])
        py_max = jnp.maximum(jnp.maximum(pyi[0], pyi[1]), pyi[2])
        cput(R_PXMIN, px_min.astype(jnp.float32))
        cput(R_PXMAX, px_max.astype(jnp.float32))
        cput(R_PYMIN, py_min.astype(jnp.float32))
        cput(R_PYMAX, py_max.astype(jnp.float32))
        cput(R_VALID, jnp.where(valid, 1.0, 0.0))

        xmin_acc = jnp.minimum(xmin_acc, jnp.where(valid, px_min, big_i))
        xmax_acc = jnp.maximum(xmax_acc, jnp.where(valid, px_max, -big_i))
        ymin_acc = jnp.minimum(ymin_acc, jnp.where(valid, py_min, big_i))
        ymax_acc = jnp.maximum(ymax_acc, jnp.where(valid, py_max, -big_i))
        anyv_acc = anyv_acc | valid.astype(jnp.int32)

    x_min = jnp.min(xmin_acc)
    x_max = jnp.max(xmax_acc)
    y_min = jnp.min(ymin_acc)
    y_max = jnp.max(ymax_acc)
    any_valid = jnp.max(anyv_acc) > 0
    range_x_min = jnp.maximum(x_min - lax.rem(x_min, BLK), 0)
    range_y_min = jnp.maximum(y_min - lax.rem(y_min, BLK), 0)
    range_x_max = jnp.minimum(x_max, W)
    range_y_max = jnp.minimum(y_max, H)

    iota_f = iota_i.astype(jnp.float32)

    def sweep_face(f, first):
        fi32 = _splat(f, jnp.int32)

        def cb(row):
            return plsc.load_gather(coef_v, [_splat(row * F, jnp.int32) + fi32])

        bdx, bdy, bdc = cb(R_DX), cb(R_DY), cb(R_DC)
        sdx = bdx * float(L)
        valid_s = jnp.max(cb(R_VALID)) > 0.5

        def update(j, rbase, d):
            d = jnp.where(d != d, INF_VALUE, d)
            sl = pl.ds(rbase + j * L, L)
            if first:
                best_d[sl] = d
                best_i[sl] = _splat(0, jnp.int32)
            else:
                bd = best_d[sl]
                upd = d < bd
                best_d[sl] = jnp.where(upd, d, bd)
                best_i[sl] = jnp.where(upd, fi32, best_i[sl])

        def full_path():
            bl0x, bl0y, bl0c = cb(R_L0X), cb(R_L0Y), cb(R_L0C)
            bl1x, bl1y, bl1c = cb(R_L1X), cb(R_L1Y), cb(R_L1C)
            bl2x, bl2y, bl2c = cb(R_L2X), cb(R_L2Y), cb(R_L2C)
            bpxmin, bpxmax = cb(R_PXMIN), cb(R_PXMAX)
            bpymin, bpymax = cb(R_PYMIN), cb(R_PYMAX)
            s0, s1, s2 = bl0x * float(L), bl1x * float(L), bl2x * float(L)

            def row_body(r, _):
                y = row0 + r
                yf = _splat(y.astype(jnp.float32))
                ibyf = _splat((y - lax.rem(y, BLK)).astype(jnp.float32))
                tgy = (bpymax >= ibyf) & (bpymin < ibyf + float(BLK))
                l0v = bl0x * iota_f + (bl0y * yf + bl0c)
                l1v = bl1x * iota_f + (bl1y * yf + bl1c)
                l2v = bl2x * iota_f + (bl2y * yf + bl2c)
                rv = bdx * iota_f + (bdy * yf + bdc)
                rbase = r * W

                for j in range(NCHUNK):
                    if j % 2 == 0:
                        kbxf = jnp.full(
                            (L,), float((j * L) - (j * L) % BLK), jnp.float32)
                        tgxy = ((bpxmax >= kbxf)
                                & (bpxmin < kbxf + float(BLK))
                                & (tgy & bvalid))
                    inside = ((l0v >= 0.0) & (l1v >= 0.0)
                              & ((l2v >= 0.0) & tgxy))
                    d = jnp.where(inside, rv, INF_VALUE + rv)
                    update(j, rbase, d)
                    if j != NCHUNK - 1:
                        l0v = l0v + s0
                        l1v = l1v + s1
                        l2v = l2v + s2
                        rv = rv + sdx
                return 0

            lax.fori_loop(0, ROWS_PW, row_body, 0)

        def fast_path():
            def row_body(r, _):
                y = row0 + r
                yf = _splat(y.astype(jnp.float32))
                rv = bdx * iota_f + (bdy * yf + bdc)
                rbase = r * W

                for j in range(NCHUNK):
                    update(j, rbase, INF_VALUE + rv)
                    if j != NCHUNK - 1:
                        rv = rv + sdx
                return 0

            lax.fori_loop(0, ROWS_PW, row_body, 0)

        del valid_s, fast_path
        bvalid = cb(R_VALID) > 0.5
        full_path()

    sweep_face(jnp.int32(0), True)

    def face_body(f, _):
        sweep_face(f, False)
        return 0

    lax.fori_loop(1, F, face_body, 0)

    rxminf = _splat(range_x_min.astype(jnp.float32))
    rxmaxf = _splat(range_x_max.astype(jnp.float32))
    ryminf = _splat(range_y_min.astype(jnp.float32))
    rymaxf = _splat(range_y_max.astype(jnp.float32))
    anyv_v = _splat(jnp.where(any_valid, 1.0, 0.0)) > 0.5

    def color_row(r, _):
        y = row0 + r
        yf = _splat(y.astype(jnp.float32))
        ibyf = _splat((y - lax.rem(y, BLK)).astype(jnp.float32))
        procy = (ibyf >= ryminf) & (ibyf < rymaxf)
        mrow = procy & anyv_v
        rbase = r * W

        def color_chunk(j, _):
            xi = iota_i + j * L
            xf = xi.astype(jnp.float32)
            kbxf = (xi - lax.rem(xi, BLK)).astype(jnp.float32)
            sl = pl.ds(rbase + j * L, L)
            bd = best_d[sl]
            bi = best_i[sl]
            vis = bd < LOWER_INF
            procx = (kbxf >= rxminf) & (kbxf < rxmaxf)
            covered = vis & procx & mrow
            for ch in range(3):
                cx = plsc.load_gather(
                    coef_v, [_splat((R_CX + ch) * F, jnp.int32) + bi])
                cy = plsc.load_gather(
                    coef_v, [_splat((R_CY + ch) * F, jnp.int32) + bi])
                cc = plsc.load_gather(
                    coef_v, [_splat((R_CC + ch) * F, jnp.int32) + bi])
                pix = cx * xf + cy * yf + cc
                img_v[pl.ds(ch * PXW + rbase + j * L, L)] = (
                    jnp.where(covered, pix, 0.0))
            msk_v[sl] = jnp.where(covered, 1.0, 0.0)
            return 0

        return lax.fori_loop(0, NCHUNK, color_chunk, 0)

    lax.fori_loop(0, ROWS_PW, color_row, 0)

    for ch in range(3):
        pltpu.sync_copy(
            img_v.at[pl.ds(ch * PXW, PXW)],
            img_h.at[pl.ds(((b * 3 + ch) * H + row0) * W, PXW)])
    pltpu.sync_copy(msk_v, msk_h.at[pl.ds((b * H + row0) * W, PXW)])


@jax.jit
def _rasterize(pt2p, colp, pt3p, nrmp, rtp, facep):
    mesh = plsc.VectorSubcoreMesh(core_axis_name="c", subcore_axis_name="s")
    run = functools.partial(
        pl.kernel,
        out_type=[
            jax.ShapeDtypeStruct((B * 3 * H * W,), jnp.float32),
            jax.ShapeDtypeStruct((B * H * W,), jnp.float32),
        ],
        mesh=mesh,
        compiler_params=pltpu.CompilerParams(needs_layout_passes=False),
        scratch_types=[
            pltpu.VMEM((B * 3 * VP,), jnp.float32),
            pltpu.VMEM((B * 3 * VP,), jnp.float32),
            pltpu.VMEM((B * 3 * VP,), jnp.float32),
            pltpu.VMEM((B * 3 * F,), jnp.float32),
            pltpu.VMEM((B * L,), jnp.float32),
            pltpu.VMEM((3 * F,), jnp.int32),
            pltpu.VMEM((NCOEF * F,), jnp.float32),
            pltpu.VMEM((PXW,), jnp.float32),
            pltpu.VMEM((PXW,), jnp.int32),
            pltpu.VMEM((3 * PXW,), jnp.float32),
            pltpu.VMEM((PXW,), jnp.float32),
        ],
    )(_rasterize_body)
    img_flat, msk_flat = run(pt2p, colp, pt3p, nrmp, rtp, facep)
    image = img_flat.reshape(B, 3, H, W)
    mask = msk_flat.reshape(B, H, W)
    return image, mask


def kernel(pt_2d, color, pt_3d, normal, R, T, face):
    pad = ((0, 0), (0, 0), (0, VP - V))
    pt2p = jnp.pad(pt_2d, pad).reshape(-1)
    colp = jnp.pad(color, pad).reshape(-1)
    pt3p = jnp.pad(pt_3d, pad).reshape(-1)
    rtp = jnp.concatenate(
        [R.reshape(B, 9), T.reshape(B, 3), jnp.zeros((B, 4), jnp.float32)],
        axis=1).reshape(-1)
    return _rasterize(pt2p, colp, pt3p, normal.reshape(-1), rtp,
                      face.astype(jnp.int32).reshape(-1))
